# Initial kernel scaffold; baseline (speedup 1.0000x reference)
import jax
import jax.numpy as jnp
from jax.experimental import pallas as pl


def kernel(mask, pc):
    raise NotImplementedError("write your pallas kernel here")


# speedup vs baseline: 29.0303x; 29.0303x over previous
import functools

import jax
import jax.numpy as jnp
from jax import lax
from jax.experimental import pallas as pl
from jax.experimental.pallas import tpu as pltpu
from jax.experimental.pallas import tpu_sc as plsc

N = 8192
KNN = 8
C = 128
MAX_R = 1.0


ROWS = 256
NBLK = N // ROWS


def _knn_body(pc_blk_ref, pcT_ref, idx_ref):
    pc_blk = pc_blk_ref[...]
    pcT = pcT_ref[...]
    dot = lax.dot_general(pc_blk, pcT, (((1,), (0,)), ((), ())),
                          preferred_element_type=jnp.float32)
    d2r = jnp.sum(pc_blk * pc_blk, axis=1, keepdims=True)
    d2c = jnp.sum(pcT * pcT, axis=0, keepdims=True)
    dist = (d2r + d2c) - 2.0 * dot

    iota = lax.broadcasted_iota(jnp.int32, (ROWS, N), 1)
    bigi = jnp.int32(2**30)
    idxs, mins = [], []
    for k in range(KNN):
        m = jnp.min(dist, axis=1, keepdims=True)
        amin = jnp.min(jnp.where(dist == m, iota, bigi), axis=1,
                       keepdims=True)
        idxs.append(amin)
        mins.append(m)
        if k < KNN - 1:
            dist = jnp.where(iota == amin, jnp.float32(jnp.inf), dist)
    idx = jnp.concatenate(idxs, axis=1)
    dk = jnp.concatenate(mins, axis=1)
    idx = jnp.where(dk > MAX_R, jnp.broadcast_to(idx[:, 0:1], (ROWS, KNN)),
                    idx)
    idx_ref[...] = idx


def _knn(pc_pad, pcT):
    return pl.pallas_call(
        _knn_body,
        grid=(NBLK,),
        in_specs=[
            pl.BlockSpec((ROWS, 8), lambda i: (i, 0)),
            pl.BlockSpec((8, N), lambda i: (0, 0)),
        ],
        out_specs=pl.BlockSpec((ROWS, KNN), lambda i: (i, 0)),
        out_shape=jax.ShapeDtypeStruct((N, KNN), jnp.int32),
    )(pc_pad, pcT)


NC, NS, LANES = 2, 16, 16
NW = NC * NS
PTS_W = N // NW
PAIRS_W = PTS_W * KNN
CHUNK_PTS = 16
CHUNK_PAIRS = CHUNK_PTS * KNN
NCHUNK = PTS_W // CHUNK_PTS

@functools.cache
def _build_sc_loss():
    mesh = plsc.VectorSubcoreMesh(core_axis_name="c", subcore_axis_name="s")
    return pl.kernel(
        _sc_loss_body,
        out_type=jax.ShapeDtypeStruct((N * KNN,), jnp.float32),
        mesh=mesh,
        compiler_params=pltpu.CompilerParams(needs_layout_passes=False),
        scratch_types=[
            pltpu.VMEM((PAIRS_W,), jnp.int32),
            pltpu.VMEM((PTS_W, C), jnp.float32),
            pltpu.VMEM((CHUNK_PAIRS, C), jnp.float32),
            pltpu.VMEM((PAIRS_W,), jnp.float32),
            pltpu.VMEM((LANES, LANES), jnp.float32),
            pltpu.SemaphoreType.DMA,
        ],
    )


def _sc_loss_body(mask_hbm, idx_hbm, out_hbm, idx_v, own_v, gat_v, res_v,
                  tmp_v, sem):
    wid = lax.axis_index("s") * NC + lax.axis_index("c")
    pbase = wid * PAIRS_W
    nbase = wid * PTS_W
    pltpu.sync_copy(idx_hbm.at[pl.ds(pbase, PAIRS_W)], idx_v)
    pltpu.sync_copy(mask_hbm.at[pl.ds(nbase, PTS_W)], own_v)

    lane_iota = lax.broadcasted_iota(jnp.int32, (LANES,), 0)
    ncv = C // LANES

    def chunk_body(ci, _):
        idx_slice = idx_v.at[pl.ds(ci * CHUNK_PAIRS, CHUNK_PAIRS)]
        pltpu.async_copy(mask_hbm.at[idx_slice], gat_v, sem).wait()

        def grp_body(g, _):
            p0 = ci * CHUNK_PTS + g * 2
            own = [own_v[p0 + (j // ncv), pl.ds((j % ncv) * LANES, LANES)]
                   for j in range(2 * ncv)]
            for j in range(LANES):
                row = g * LANES + j
                pt = j // KNN
                acc = jnp.abs(own[pt * ncv] -
                              gat_v[row, pl.ds(0, LANES)])
                for c8 in range(1, ncv):
                    b = gat_v[row, pl.ds(c8 * LANES, LANES)]
                    acc = acc + jnp.abs(own[pt * ncv + c8] - b)
                tmp_v[j, :] = acc
            out_vec = plsc.load_gather(
                tmp_v, [lane_iota, jnp.zeros((LANES,), jnp.int32)])
            for c in range(1, LANES):
                out_vec = out_vec + plsc.load_gather(
                    tmp_v, [lane_iota, jnp.full((LANES,), c, jnp.int32)])
            res_v[pl.ds(ci * CHUNK_PAIRS + g * LANES, LANES)] = out_vec
            return 0

        lax.fori_loop(0, CHUNK_PAIRS // LANES, grp_body, 0)
        return 0

    lax.fori_loop(0, NCHUNK, chunk_body, 0)
    pltpu.sync_copy(res_v, out_hbm.at[pl.ds(pbase, PAIRS_W)])


def _mean_body(x_ref, o_ref):
    o_ref[...] = jnp.sum(x_ref[...]).reshape(1, 1) * (1.0 / (N * KNN))


def _mean(per_pair):
    return pl.pallas_call(
        _mean_body,
        out_shape=jax.ShapeDtypeStruct((1, 1), jnp.float32),
    )(per_pair.reshape(N * KNN // C, C))


def kernel(mask, pc):
    maskf = mask[0]
    pcf = pc[0]
    pc_pad = jnp.pad(pcf, ((0, 0), (0, 5)))
    pcT = pc_pad.T
    nn_idx = _knn(pc_pad, pcT)
    per_pair = _build_sc_loss()(maskf, nn_idx.reshape(-1))
    per_point = per_pair.reshape(1, N, KNN)
    smooth = _mean(per_pair).reshape(())
    return (smooth, per_point)
